# baseline (speedup 1.0000x reference)
import functools

import jax
import jax.numpy as jnp
from jax import lax
from jax.experimental import pallas as pl
from jax.experimental.pallas import tpu as pltpu
from jax.experimental.pallas import tpu_sc as plsc

N = 10000
E = 320000
D = 128

NC = 2
NS = 16
W = 88

EDGES_PER_TILE = E // (NC * NS)
NWIN = EDGES_PER_TILE // W
TAILW = EDGES_PER_TILE - NWIN * W
N_PAD = 10240
CNT_PER_TILE = N_PAD // NS
ZTILES = 10
ZROWS = N // ZTILES

DEPTH = 4

_mesh = plsc.VectorSubcoreMesh(core_axis_name="c", subcore_axis_name="s")


@functools.partial(
    pl.kernel,
    out_type=(jax.ShapeDtypeStruct((NC, N, D), jnp.float32),
              jax.ShapeDtypeStruct((NC, N_PAD), jnp.float32)),
    mesh=_mesh,
    scratch_types=[
        pltpu.VMEM_SHARED((N, D), jnp.float32),
        pltpu.VMEM_SHARED((N_PAD,), jnp.float32),
        pltpu.VMEM((W,), jnp.float32),
        [pltpu.VMEM((W,), jnp.int32) for _ in range(2 * DEPTH)],
        [pltpu.VMEM((W,), jnp.int32) for _ in range(2 * DEPTH)],
        [pltpu.VMEM((W, D), jnp.float32) for _ in range(DEPTH)],
        pltpu.VMEM((TAILW,), jnp.int32),
        [pltpu.SemaphoreType.DMA for _ in range(2 * DEPTH)],
        [pltpu.SemaphoreType.DMA for _ in range(DEPTH)],
        [pltpu.SemaphoreType.DMA for _ in range(DEPTH)],
        [pltpu.SemaphoreType.DMA for _ in range(DEPTH)],
    ],
)
def _sc_agg(src_hbm, dst_hbm, x_hbm, zero_hbm, ones_hbm, zcnt_hbm,
            out_hbm, cnt_hbm,
            acc_sh, cnt_sh, ones_v, src_vs, dst_vs, rows_vs, tail_dst,
            isems, gsems, ssems, csems):
    c = lax.axis_index("c")
    s = lax.axis_index("s")
    tid = c * NS + s
    base = tid * EDGES_PER_TILE
    NI = 2 * DEPTH

    @pl.when(s < ZTILES)
    def _():
        pltpu.sync_copy(zero_hbm, acc_sh.at[pl.ds(s * ZROWS, ZROWS)])

    pltpu.sync_copy(zcnt_hbm, cnt_sh.at[pl.ds(s * CNT_PER_TILE, CNT_PER_TILE)])
    pltpu.sync_copy(ones_hbm, ones_v)
    plsc.subcore_barrier()

    def load_idx(m, w):
        off = base + w * W
        pltpu.async_copy(src_hbm.at[pl.ds(off, W)], src_vs[m], isems[m])
        pltpu.async_copy(dst_hbm.at[pl.ds(off, W)], dst_vs[m], isems[m])

    def wait_idx(m):
        pltpu.make_async_copy(src_hbm.at[pl.ds(0, W)], src_vs[m],
                              isems[m]).wait()
        pltpu.make_async_copy(dst_hbm.at[pl.ds(0, W)], dst_vs[m],
                              isems[m]).wait()

    def gather(k, m):
        pltpu.async_copy(x_hbm.at[src_vs[m]], rows_vs[k], gsems[k])

    def wait_gather(k, m):
        pltpu.make_async_copy(x_hbm.at[src_vs[m]], rows_vs[k],
                              gsems[k]).wait()

    def scatter(k, m):
        pltpu.async_copy(rows_vs[k], acc_sh.at[dst_vs[m]], ssems[k], add=True)

    def wait_scatter(k, m):
        pltpu.make_async_copy(rows_vs[k], acc_sh.at[dst_vs[m]],
                              ssems[k]).wait()

    def cscatter(k, m):
        pltpu.async_copy(ones_v, cnt_sh.at[dst_vs[m]], csems[k], add=True)

    def wait_cscatter(k, m):
        pltpu.make_async_copy(ones_v, cnt_sh.at[dst_vs[m]], csems[k]).wait()

    for m in range(NI):
        load_idx(m, m)
    for k in range(DEPTH):
        wait_idx(k)
        gather(k, k)

    def body(i, _):
        for j in range(NI):
            k = j % DEPTH
            wait_gather(k, j)
            scatter(k, j)
            cscatter(k, j)
            wait_scatter(k, j)
            wait_cscatter(k, j)
            load_idx(j, (i + 1) * NI + j)
            m2 = (j + DEPTH) % NI
            wait_idx(m2)
            gather(k, m2)
        return _

    NROUND = NWIN // NI
    lax.fori_loop(0, NROUND - 1, body, None)

    for j in range(NI):
        k = j % DEPTH
        wait_gather(k, j)
        scatter(k, j)
        cscatter(k, j)
        wait_scatter(k, j)
        wait_cscatter(k, j)
        if j < DEPTH:
            m2 = j + DEPTH
            wait_idx(m2)
            gather(k, m2)
    for w in range(NROUND * NI, NWIN):
        load_idx(0, w)
        wait_idx(0)
        gather(0, 0)
        wait_gather(0, 0)
        scatter(0, 0)
        cscatter(0, 0)
        wait_scatter(0, 0)
        wait_cscatter(0, 0)

    toff = base + NWIN * W
    pltpu.async_copy(src_hbm.at[pl.ds(toff, TAILW)],
                     src_vs[0].at[pl.ds(0, TAILW)], isems[0])
    pltpu.async_copy(dst_hbm.at[pl.ds(toff, TAILW)], tail_dst, isems[0])
    pltpu.make_async_copy(src_hbm.at[pl.ds(0, TAILW)],
                          src_vs[0].at[pl.ds(0, TAILW)], isems[0]).wait()
    pltpu.make_async_copy(dst_hbm.at[pl.ds(0, TAILW)], tail_dst,
                          isems[0]).wait()
    pltpu.async_copy(x_hbm.at[src_vs[0].at[pl.ds(0, TAILW)]],
                     rows_vs[0].at[pl.ds(0, TAILW)], gsems[0])
    pltpu.make_async_copy(x_hbm.at[src_vs[0].at[pl.ds(0, TAILW)]],
                          rows_vs[0].at[pl.ds(0, TAILW)], gsems[0]).wait()
    pltpu.async_copy(rows_vs[0].at[pl.ds(0, TAILW)], acc_sh.at[tail_dst],
                     ssems[0], add=True)
    pltpu.make_async_copy(rows_vs[0].at[pl.ds(0, TAILW)], acc_sh.at[tail_dst],
                          ssems[0]).wait()
    pltpu.async_copy(ones_v.at[pl.ds(0, TAILW)], cnt_sh.at[tail_dst],
                     csems[0], add=True)
    pltpu.make_async_copy(ones_v.at[pl.ds(0, TAILW)], cnt_sh.at[tail_dst],
                          csems[0]).wait()

    plsc.subcore_barrier()
    @pl.when(s < ZTILES)
    def _():
        pltpu.sync_copy(acc_sh.at[pl.ds(s * ZROWS, ZROWS)],
                        out_hbm.at[c, pl.ds(s * ZROWS, ZROWS)])

    pltpu.sync_copy(cnt_sh.at[pl.ds(s * CNT_PER_TILE, CNT_PER_TILE)],
                    cnt_hbm.at[c, pl.ds(s * CNT_PER_TILE, CNT_PER_TILE)])


def _tc_dense_body(parts_ref, cnts_ref, x_ref, wlt_ref, wrt_ref, bl_ref,
                   g_ref, b_ref, o_ref):
    cnt = jnp.maximum(cnts_ref[0] + cnts_ref[1], 1.0)
    a = (parts_ref[0] + parts_ref[1]) * (1.0 / cnt)[:, None]
    y = (jnp.dot(a, wlt_ref[:], preferred_element_type=jnp.float32)
         + jnp.dot(x_ref[:], wrt_ref[:], preferred_element_type=jnp.float32)
         + bl_ref[:])
    mean = jnp.mean(y, axis=0, keepdims=True)
    var = jnp.mean((y - mean) ** 2, axis=0, keepdims=True)
    yn = (y - mean) * (lax.rsqrt(var + 1e-5) * g_ref[:]) + b_ref[:]
    o_ref[:] = jnp.where(yn >= 0, yn, 0.01 * yn)


_tc_dense = pl.pallas_call(
    _tc_dense_body,
    out_shape=jax.ShapeDtypeStruct((N, D), jnp.float32),
)


def kernel(x, edge_index, Wl0, bl0, Wr0, g0, b0, Wl1, bl1, Wr1, g1, b1,
           Wl2, bl2, Wr2, g2, b2, Wl3, bl3, Wr3, g3, b3):
    params = ((Wl0, bl0, Wr0, g0, b0), (Wl1, bl1, Wr1, g1, b1),
              (Wl2, bl2, Wr2, g2, b2), (Wl3, bl3, Wr3, g3, b3))
    src = edge_index[0].astype(jnp.int32)
    dst = edge_index[1].astype(jnp.int32)
    zero_rows = jnp.zeros((ZROWS, D), jnp.float32)
    zero_cnt = jnp.zeros((CNT_PER_TILE,), jnp.float32)
    ones_w = jnp.ones((W,), jnp.float32)

    for Wl, bl, Wr, g, b in params:
        parts, cnts_pad = _sc_agg(src, dst, x, zero_rows, ones_w, zero_cnt)
        cnts = cnts_pad[:, :N]
        x = _tc_dense(parts, cnts, x, Wl.T, Wr.T,
                      bl.reshape(1, D), g.reshape(1, D), b.reshape(1, D))
    return x


# speedup vs baseline: 1.0339x; 1.0339x over previous
import functools

import jax
import jax.numpy as jnp
from jax import lax
from jax.experimental import pallas as pl
from jax.experimental.pallas import tpu as pltpu
from jax.experimental.pallas import tpu_sc as plsc

N = 10000
E = 320000
D = 128

NC = 2
NS = 16
W = 88

EDGES_PER_TILE = E // (NC * NS)
NWIN = EDGES_PER_TILE // W
TAILW = EDGES_PER_TILE - NWIN * W
N_PAD = 10240
CNT_PER_TILE = N_PAD // NS
ZTILES = 10
ZROWS = N // ZTILES

DEPTH = 4

CW = 80
CNWIN = EDGES_PER_TILE // CW
NITER = CNWIN // DEPTH

_mesh = plsc.VectorSubcoreMesh(core_axis_name="c", subcore_axis_name="s")


def _fill_idx(dst_buf, src_buf, off):
    for j in range(CW // 16):
        dst_buf[pl.ds(j * 16, 16)] = src_buf[pl.ds(off + j * 16, 16)]


@functools.partial(
    pl.kernel,
    out_type=jax.ShapeDtypeStruct((NC, N, D), jnp.float32),
    mesh=_mesh,
    scratch_types=[
        pltpu.VMEM_SHARED((N, D), jnp.float32),
        [pltpu.VMEM((W,), jnp.int32) for _ in range(2 * DEPTH)],
        [pltpu.VMEM((W,), jnp.int32) for _ in range(2 * DEPTH)],
        [pltpu.VMEM((W, D), jnp.float32) for _ in range(DEPTH)],
        pltpu.VMEM((TAILW,), jnp.int32),
        [pltpu.SemaphoreType.DMA for _ in range(2 * DEPTH)],
        [pltpu.SemaphoreType.DMA for _ in range(DEPTH)],
        [pltpu.SemaphoreType.DMA for _ in range(DEPTH)],
    ],
)
def _sc_agg(src_hbm, dst_hbm, x_hbm, zero_hbm, out_hbm,
            acc_sh, src_vs, dst_vs, rows_vs, tail_dst, isems, gsems, ssems):
    c = lax.axis_index("c")
    s = lax.axis_index("s")
    tid = c * NS + s
    base = tid * EDGES_PER_TILE
    NI = 2 * DEPTH

    @pl.when(s < ZTILES)
    def _():
        pltpu.sync_copy(zero_hbm, acc_sh.at[pl.ds(s * ZROWS, ZROWS)])

    plsc.subcore_barrier()

    def load_idx(m, w):
        off = base + w * W
        pltpu.async_copy(src_hbm.at[pl.ds(off, W)], src_vs[m], isems[m])
        pltpu.async_copy(dst_hbm.at[pl.ds(off, W)], dst_vs[m], isems[m])

    def wait_idx(m):
        pltpu.make_async_copy(src_hbm.at[pl.ds(0, W)], src_vs[m],
                              isems[m]).wait()
        pltpu.make_async_copy(dst_hbm.at[pl.ds(0, W)], dst_vs[m],
                              isems[m]).wait()

    def gather(k, m):
        pltpu.async_copy(x_hbm.at[src_vs[m]], rows_vs[k], gsems[k])

    def wait_gather(k, m):
        pltpu.make_async_copy(x_hbm.at[src_vs[m]], rows_vs[k],
                              gsems[k]).wait()

    def scatter(k, m):
        pltpu.async_copy(rows_vs[k], acc_sh.at[dst_vs[m]], ssems[k], add=True)

    def wait_scatter(k, m):
        pltpu.make_async_copy(rows_vs[k], acc_sh.at[dst_vs[m]],
                              ssems[k]).wait()

    for m in range(NI):
        load_idx(m, m)
    for k in range(DEPTH):
        wait_idx(k)
        gather(k, k)

    def body(i, _):
        for j in range(NI):
            k = j % DEPTH
            wait_gather(k, j)
            scatter(k, j)
            wait_scatter(k, j)
            load_idx(j, (i + 1) * NI + j)
            m2 = (j + DEPTH) % NI
            wait_idx(m2)
            gather(k, m2)
        return _

    NROUND = NWIN // NI
    lax.fori_loop(0, NROUND - 1, body, None)

    for j in range(NI):
        k = j % DEPTH
        wait_gather(k, j)
        scatter(k, j)
        wait_scatter(k, j)
        if j < DEPTH:
            m2 = j + DEPTH
            wait_idx(m2)
            gather(k, m2)
    for w in range(NROUND * NI, NWIN):
        load_idx(0, w)
        wait_idx(0)
        gather(0, 0)
        wait_gather(0, 0)
        scatter(0, 0)
        wait_scatter(0, 0)

    toff = base + NWIN * W
    pltpu.async_copy(src_hbm.at[pl.ds(toff, TAILW)],
                     src_vs[0].at[pl.ds(0, TAILW)], isems[0])
    pltpu.async_copy(dst_hbm.at[pl.ds(toff, TAILW)], tail_dst, isems[0])
    pltpu.make_async_copy(src_hbm.at[pl.ds(0, TAILW)],
                          src_vs[0].at[pl.ds(0, TAILW)], isems[0]).wait()
    pltpu.make_async_copy(dst_hbm.at[pl.ds(0, TAILW)], tail_dst,
                          isems[0]).wait()
    pltpu.async_copy(x_hbm.at[src_vs[0].at[pl.ds(0, TAILW)]],
                     rows_vs[0].at[pl.ds(0, TAILW)], gsems[0])
    pltpu.make_async_copy(x_hbm.at[src_vs[0].at[pl.ds(0, TAILW)]],
                          rows_vs[0].at[pl.ds(0, TAILW)], gsems[0]).wait()
    pltpu.async_copy(rows_vs[0].at[pl.ds(0, TAILW)], acc_sh.at[tail_dst],
                     ssems[0], add=True)
    pltpu.make_async_copy(rows_vs[0].at[pl.ds(0, TAILW)], acc_sh.at[tail_dst],
                          ssems[0]).wait()

    plsc.subcore_barrier()
    @pl.when(s < ZTILES)
    def _():
        pltpu.sync_copy(acc_sh.at[pl.ds(s * ZROWS, ZROWS)],
                        out_hbm.at[c, pl.ds(s * ZROWS, ZROWS)])


@functools.partial(
    pl.kernel,
    out_type=jax.ShapeDtypeStruct((NC, N_PAD), jnp.float32),
    mesh=_mesh,
    scratch_types=[
        pltpu.VMEM((EDGES_PER_TILE,), jnp.int32),
        pltpu.VMEM((CW,), jnp.float32),
        pltpu.VMEM_SHARED((N_PAD,), jnp.float32),
        [pltpu.VMEM((CW,), jnp.int32) for _ in range(DEPTH)],
        [pltpu.SemaphoreType.DMA for _ in range(DEPTH)],
    ],
)
def _sc_counts(dst_hbm, zero_hbm, out_hbm, dst_all, ones_v, cnt_sh,
               dst_vs, ssems):
    c = lax.axis_index("c")
    s = lax.axis_index("s")
    tid = c * NS + s
    base = tid * EDGES_PER_TILE

    for k in range(CW // 16):
        ones_v[pl.ds(k * 16, 16)] = jnp.ones((16,), jnp.float32)

    pltpu.sync_copy(dst_hbm.at[pl.ds(base, EDGES_PER_TILE)], dst_all)
    pltpu.sync_copy(zero_hbm, cnt_sh.at[pl.ds(s * CNT_PER_TILE, CNT_PER_TILE)])
    plsc.subcore_barrier()

    for k in range(DEPTH):
        _fill_idx(dst_vs[k], dst_all, k * CW)
        pltpu.async_copy(ones_v, cnt_sh.at[dst_vs[k]], ssems[k], add=True)

    def body(i, _):
        for k in range(DEPTH):
            w_next = (i + 1) * DEPTH + k
            pltpu.make_async_copy(ones_v, cnt_sh.at[dst_vs[k]],
                                  ssems[k]).wait()
            _fill_idx(dst_vs[k], dst_all, w_next * CW)
            pltpu.async_copy(ones_v, cnt_sh.at[dst_vs[k]], ssems[k], add=True)
        return _

    lax.fori_loop(0, NITER - 1, body, None)
    for k in range(DEPTH):
        pltpu.make_async_copy(ones_v, cnt_sh.at[dst_vs[k]], ssems[k]).wait()
    for w in range(NITER * DEPTH, CNWIN):
        _fill_idx(dst_vs[0], dst_all, w * CW)
        pltpu.async_copy(ones_v, cnt_sh.at[dst_vs[0]], ssems[0], add=True)
        pltpu.make_async_copy(ones_v, cnt_sh.at[dst_vs[0]], ssems[0]).wait()
    plsc.subcore_barrier()

    pltpu.sync_copy(cnt_sh.at[pl.ds(s * CNT_PER_TILE, CNT_PER_TILE)],
                    out_hbm.at[c, pl.ds(s * CNT_PER_TILE, CNT_PER_TILE)])


def _tc_dense_body(parts_ref, cnts_ref, x_ref, wlt_ref, wrt_ref, bl_ref,
                   g_ref, b_ref, o_ref):
    cnt = jnp.maximum(cnts_ref[0] + cnts_ref[1], 1.0)
    a = (parts_ref[0] + parts_ref[1]) * (1.0 / cnt)[:, None]
    y = (jnp.dot(a, wlt_ref[:], preferred_element_type=jnp.float32)
         + jnp.dot(x_ref[:], wrt_ref[:], preferred_element_type=jnp.float32)
         + bl_ref[:])
    mean = jnp.mean(y, axis=0, keepdims=True)
    var = jnp.mean((y - mean) ** 2, axis=0, keepdims=True)
    yn = (y - mean) * (lax.rsqrt(var + 1e-5) * g_ref[:]) + b_ref[:]
    o_ref[:] = jnp.where(yn >= 0, yn, 0.01 * yn)


_tc_dense = pl.pallas_call(
    _tc_dense_body,
    out_shape=jax.ShapeDtypeStruct((N, D), jnp.float32),
)


def kernel(x, edge_index, Wl0, bl0, Wr0, g0, b0, Wl1, bl1, Wr1, g1, b1,
           Wl2, bl2, Wr2, g2, b2, Wl3, bl3, Wr3, g3, b3):
    params = ((Wl0, bl0, Wr0, g0, b0), (Wl1, bl1, Wr1, g1, b1),
              (Wl2, bl2, Wr2, g2, b2), (Wl3, bl3, Wr3, g3, b3))
    src = edge_index[0].astype(jnp.int32)
    dst = edge_index[1].astype(jnp.int32)
    zero_rows = jnp.zeros((ZROWS, D), jnp.float32)
    zero_cnt = jnp.zeros((CNT_PER_TILE,), jnp.float32)

    cnts = _sc_counts(dst, zero_cnt)[:, :N]
    for Wl, bl, Wr, g, b in params:
        parts = _sc_agg(src, dst, x, zero_rows)
        x = _tc_dense(parts, cnts, x, Wl.T, Wr.T,
                      bl.reshape(1, D), g.reshape(1, D), b.reshape(1, D))
    return x
